# baseline (speedup 1.0000x reference)
import functools

import jax
import jax.numpy as jnp
from jax import lax
from jax.experimental import pallas as pl
from jax.experimental.pallas import tpu as pltpu
from jax.experimental.pallas import tpu_sc as plsc

N = 10000
E = 320000
D = 128
H = 32
G = 128

NC = 2
NS = 16
NW = NC * NS
NP = 10240
RPT = NP // NS
CHUNK = 128
E_PAD = 327680
CPW = E_PAD // (NW * CHUNK)
GPW = E_PAD // (NW * 16)
NPH = 8
HQ = H // NPH
NQ = NW // NPH
EQ = E_PAD // NQ
CPB = 80


@functools.cache
def _get_deg_kernel():
  mesh = plsc.VectorSubcoreMesh(
      core_axis_name="c", subcore_axis_name="s", num_cores=NC)

  @functools.partial(
      pl.kernel,
      out_type=jax.ShapeDtypeStruct((NW * NP,), jnp.float32),
      mesh=mesh,
      compiler_params=pltpu.CompilerParams(needs_layout_passes=False),
      scratch_types=[
          pltpu.VMEM((CPW * CHUNK,), jnp.int32),
          pltpu.VMEM((NP,), jnp.float32),
      ],
  )
  def deg_kernel(pk_hbm, out_hbm, pk_v, acc_v):
    c = lax.axis_index("c")
    s = lax.axis_index("s")
    wid = s * NC + c

    @plsc.parallel_loop(0, NP // 16, unroll=4)
    def _(i):
      acc_v[pl.ds(i * 16, 16)] = jnp.zeros((16,), jnp.float32)

    pltpu.sync_copy(pk_hbm.at[pl.ds(wid * CPW * CHUNK, CPW * CHUNK)], pk_v)

    ones16 = jnp.full((16,), 1.0, jnp.float32)

    @plsc.parallel_loop(0, CPW, unroll=2)
    def _(r):
      for j in range(CHUNK // 16):
        w = pk_v[pl.ds(r * CHUNK + j * 16, 16)]
        plsc.addupdate_scatter(acc_v, [jnp.right_shift(w, 14)], ones16)

    pltpu.sync_copy(acc_v, out_hbm.at[pl.ds(wid * NP, NP)])

  return deg_kernel


@functools.cache
def _get_msg_kernel():
  mesh = plsc.VectorSubcoreMesh(
      core_axis_name="c", subcore_axis_name="s", num_cores=NC)

  @functools.partial(
      pl.kernel,
      out_type=jax.ShapeDtypeStruct((NPH * NQ * HQ * NP,), jnp.float32),
      mesh=mesh,
      compiler_params=pltpu.CompilerParams(needs_layout_passes=False),
      scratch_types=[pltpu.VMEM((CPB * CHUNK,), jnp.int32),
                     pltpu.VMEM((CPB * CHUNK,), jnp.int32)]
      + [pltpu.VMEM((N,), jnp.float32) for _ in range(HQ)]
      + [pltpu.VMEM((NP,), jnp.float32) for _ in range(HQ)]
      + [pltpu.SemaphoreType.DMA, pltpu.SemaphoreType.DMA],
  )
  def msg_kernel(pk_hbm, yt_hbm, out_hbm, pk_v, pk_w, *bufs):
    y_f = bufs[:HQ]
    a_f = bufs[HQ:2 * HQ]
    sems = bufs[2 * HQ:]
    c = lax.axis_index("c")
    s = lax.axis_index("s")
    wid = s * NC + c
    ph = wid % NPH
    q = wid // NPH

    for f in range(HQ):
      pltpu.sync_copy(yt_hbm.at[ph * HQ + f], y_f[f])

    @plsc.parallel_loop(0, NP // 16, unroll=4)
    def _(i):
      z = jnp.zeros((16,), jnp.float32)
      for f in range(HQ):
        a_f[f][pl.ds(i * 16, 16)] = z

    NBLK = EQ // (CPB * CHUNK)
    pks = (pk_v, pk_w)
    cp = pltpu.async_copy(
        pk_hbm.at[pl.ds(q * EQ, CPB * CHUNK)], pks[0], sems[0])
    for blk in range(NBLK):
      cp.wait()
      if blk + 1 < NBLK:
        cp = pltpu.async_copy(
            pk_hbm.at[pl.ds(q * EQ + (blk + 1) * CPB * CHUNK, CPB * CHUNK)],
            pks[(blk + 1) % 2], sems[(blk + 1) % 2])
      pkb = pks[blk % 2]

      @plsc.parallel_loop(0, CPB, unroll=3)
      def _(r, pkb=pkb):
        for sub in range(CHUNK // 16):
          w = pkb[pl.ds(r * CHUNK + sub * 16, 16)]
          dst = jnp.right_shift(w, 14)
          src = jnp.bitwise_and(w, 16383)
          for f in range(HQ):
            vals = plsc.load_gather(y_f[f], [src])
            plsc.addupdate_scatter(a_f[f], [dst], vals)

    for f in range(HQ):
      pltpu.sync_copy(
          a_f[f], out_hbm.at[pl.ds(((q * NPH + ph) * HQ + f) * NP, NP)])

  return msg_kernel


def _scale_body(x_ref, w_ref, degp_ref, yt_ref, dinv_ref):
  xw_t = lax.dot_general(w_ref[...], x_ref[...], (((0,), (1,)), ((), ())),
                         preferred_element_type=jnp.float32)
  deg = jnp.sum(degp_ref[...], axis=0, keepdims=True) + 1.0
  dinv = lax.rsqrt(deg)
  dn = dinv[:, :N]
  yt_ref[...] = xw_t * dn
  dinv_ref[...] = dn


_scale_call = pl.pallas_call(
    _scale_body,
    out_shape=(jax.ShapeDtypeStruct((H, N), jnp.float32),
               jax.ShapeDtypeStruct((1, N), jnp.float32)),
)


def _final_body(p_ref, yt_ref, dinv_ref, bi_ref, b1_ref, wout_ref, bout_ref,
                out_ref):
  p = p_ref[...]
  pm = p[0:H]
  for w in range(1, NQ):
    pm = pm + p[w * H:(w + 1) * H]
  smsg = pm[:, :N] + yt_ref[...]
  h = jnp.tanh(smsg * dinv_ref[...] + b1_ref[...])
  bi = bi_ref[...]
  gids = lax.broadcasted_iota(jnp.int32, (G, 1), 0)
  m = (gids == bi).astype(jnp.float32)
  dims = (((1,), (1,)), ((), ()))
  sum_pool = lax.dot_general(h, m, dims, preferred_element_type=jnp.float32)
  ones_n = jnp.full((1, N), 1.0, jnp.float32)
  cnt = lax.dot_general(ones_n, m, dims, preferred_element_type=jnp.float32)
  mean_pool = sum_pool / jnp.maximum(cnt, 1.0)
  wa = wout_ref[:, 0:H]
  wb = wout_ref[:, H:2 * H]
  out = (jnp.dot(wa, sum_pool, preferred_element_type=jnp.float32)
         + jnp.dot(wb, mean_pool, preferred_element_type=jnp.float32)
         + bout_ref[...])
  out_ref[...] = out


_final_call = pl.pallas_call(
    _final_body,
    out_shape=jax.ShapeDtypeStruct((1, G), jnp.float32),
)


def kernel(x, edge_index, batch_index, W1, b1, Wout, bout):
  x = x.astype(jnp.float32)
  src = edge_index[0].astype(jnp.int32)
  dst = edge_index[1].astype(jnp.int32)
  npad = E_PAD - E
  pad_ids = jnp.arange(npad, dtype=jnp.int32)
  pad_src = pad_ids % N
  pad_dst = N + pad_ids % (NP - N)
  srcp = jnp.concatenate([src, pad_src])
  dstp = jnp.concatenate([dst, pad_dst])
  pk = dstp * 16384 + srcp

  degp = _get_deg_kernel()(pk).reshape(NW, NP)
  yt, dinv = _scale_call(x, W1.astype(jnp.float32), degp)
  p = _get_msg_kernel()(pk, yt)
  p2 = p.reshape(NQ * H, NP)

  bi = batch_index.astype(jnp.int32).reshape(1, N)
  out = _final_call(p2, yt, dinv, bi,
                    b1.astype(jnp.float32).reshape(H, 1),
                    Wout.astype(jnp.float32).reshape(1, 2 * H),
                    bout.astype(jnp.float32).reshape(1, 1))
  return out.T


# speedup vs baseline: 1.0577x; 1.0577x over previous
import functools

import jax
import jax.numpy as jnp
from jax import lax
from jax.experimental import pallas as pl
from jax.experimental.pallas import tpu as pltpu
from jax.experimental.pallas import tpu_sc as plsc

N = 10000
E = 320000
D = 128
H = 32
G = 128

NC = 2
NS = 16
NW = NC * NS
NP = 10240
RPT = NP // NS
CHUNK = 128
E_PAD = 327680
CPW = E_PAD // (NW * CHUNK)
GPW = E_PAD // (NW * 16)
NPH = 8
HQ = H // NPH
NQ = NW // NPH
EQ = E_PAD // NQ
CPB = 80


@functools.cache
def _get_deg_kernel():
  mesh = plsc.VectorSubcoreMesh(
      core_axis_name="c", subcore_axis_name="s", num_cores=NC)

  @functools.partial(
      pl.kernel,
      out_type=jax.ShapeDtypeStruct((NW * NP,), jnp.float32),
      mesh=mesh,
      compiler_params=pltpu.CompilerParams(needs_layout_passes=False),
      scratch_types=[
          pltpu.VMEM((CPW * CHUNK,), jnp.int32),
          pltpu.VMEM((NP,), jnp.float32),
      ],
  )
  def deg_kernel(pk_hbm, out_hbm, pk_v, acc_v):
    c = lax.axis_index("c")
    s = lax.axis_index("s")
    wid = s * NC + c

    @plsc.parallel_loop(0, NP // 16, unroll=4)
    def _(i):
      acc_v[pl.ds(i * 16, 16)] = jnp.zeros((16,), jnp.float32)

    pltpu.sync_copy(pk_hbm.at[pl.ds(wid * CPW * CHUNK, CPW * CHUNK)], pk_v)

    ones16 = jnp.full((16,), 1.0, jnp.float32)

    @plsc.parallel_loop(0, CPW, unroll=2)
    def _(r):
      for j in range(CHUNK // 16):
        w = pk_v[pl.ds(r * CHUNK + j * 16, 16)]
        plsc.addupdate_scatter(acc_v, [jnp.right_shift(w, 14)], ones16)

    pltpu.sync_copy(acc_v, out_hbm.at[pl.ds(wid * NP, NP)])

  return deg_kernel


@functools.cache
def _get_msg_kernel():
  mesh = plsc.VectorSubcoreMesh(
      core_axis_name="c", subcore_axis_name="s", num_cores=NC)

  @functools.partial(
      pl.kernel,
      out_type=jax.ShapeDtypeStruct((NPH * NQ * HQ * NP,), jnp.float32),
      mesh=mesh,
      compiler_params=pltpu.CompilerParams(needs_layout_passes=False),
      scratch_types=[pltpu.VMEM((CPB * CHUNK,), jnp.int32),
                     pltpu.VMEM((CPB * CHUNK,), jnp.int32)]
      + [pltpu.VMEM((N,), jnp.float32) for _ in range(HQ)]
      + [pltpu.VMEM((NP,), jnp.float32) for _ in range(HQ)]
      + [pltpu.SemaphoreType.DMA, pltpu.SemaphoreType.DMA],
  )
  def msg_kernel(pk_hbm, yt_hbm, out_hbm, pk_v, pk_w, *bufs):
    y_f = bufs[:HQ]
    a_f = bufs[HQ:2 * HQ]
    sems = bufs[2 * HQ:]
    c = lax.axis_index("c")
    s = lax.axis_index("s")
    wid = s * NC + c
    ph = wid % NPH
    q = wid // NPH

    for f in range(HQ):
      pltpu.sync_copy(yt_hbm.at[ph * HQ + f], y_f[f])

    @plsc.parallel_loop(0, NP // 16, unroll=4)
    def _(i):
      z = jnp.zeros((16,), jnp.float32)
      for f in range(HQ):
        a_f[f][pl.ds(i * 16, 16)] = z

    NBLK = EQ // (CPB * CHUNK)
    pks = (pk_v, pk_w)
    cp = pltpu.async_copy(
        pk_hbm.at[pl.ds(q * EQ, CPB * CHUNK)], pks[0], sems[0])
    for blk in range(NBLK):
      cp.wait()
      if blk + 1 < NBLK:
        cp = pltpu.async_copy(
            pk_hbm.at[pl.ds(q * EQ + (blk + 1) * CPB * CHUNK, CPB * CHUNK)],
            pks[(blk + 1) % 2], sems[(blk + 1) % 2])
      pkb = pks[blk % 2]

      @plsc.parallel_loop(0, CPB, unroll=2)
      def _(r, pkb=pkb):
        for sub in range(CHUNK // 16):
          w = pkb[pl.ds(r * CHUNK + sub * 16, 16)]
          dst = jnp.right_shift(w, 14)
          src = jnp.bitwise_and(w, 16383)
          for f in range(HQ):
            vals = plsc.load_gather(y_f[f], [src])
            plsc.addupdate_scatter(a_f[f], [dst], vals)

    for f in range(HQ):
      pltpu.sync_copy(
          a_f[f], out_hbm.at[pl.ds(((q * NPH + ph) * HQ + f) * NP, NP)])

  return msg_kernel


def _scale_body(x_ref, w_ref, degp_ref, yt_ref, dinv_ref):
  xw_t = lax.dot_general(w_ref[...], x_ref[...], (((0,), (1,)), ((), ())),
                         preferred_element_type=jnp.float32)
  deg = jnp.sum(degp_ref[...], axis=0, keepdims=True) + 1.0
  dinv = lax.rsqrt(deg)
  dn = dinv[:, :N]
  yt_ref[...] = xw_t * dn
  dinv_ref[...] = dn


_scale_call = pl.pallas_call(
    _scale_body,
    out_shape=(jax.ShapeDtypeStruct((H, N), jnp.float32),
               jax.ShapeDtypeStruct((1, N), jnp.float32)),
)


def _final_body(p_ref, yt_ref, dinv_ref, bi_ref, b1_ref, wout_ref, bout_ref,
                out_ref):
  p = p_ref[...]
  pm = p[0:H]
  for w in range(1, NQ):
    pm = pm + p[w * H:(w + 1) * H]
  smsg = pm[:, :N] + yt_ref[...]
  h = jnp.tanh(smsg * dinv_ref[...] + b1_ref[...])
  bi = bi_ref[...]
  gids = lax.broadcasted_iota(jnp.int32, (G, 1), 0)
  m = (gids == bi).astype(jnp.float32)
  dims = (((1,), (1,)), ((), ()))
  sum_pool = lax.dot_general(h, m, dims, preferred_element_type=jnp.float32)
  ones_n = jnp.full((1, N), 1.0, jnp.float32)
  cnt = lax.dot_general(ones_n, m, dims, preferred_element_type=jnp.float32)
  mean_pool = sum_pool / jnp.maximum(cnt, 1.0)
  wa = wout_ref[:, 0:H]
  wb = wout_ref[:, H:2 * H]
  out = (jnp.dot(wa, sum_pool, preferred_element_type=jnp.float32)
         + jnp.dot(wb, mean_pool, preferred_element_type=jnp.float32)
         + bout_ref[...])
  out_ref[...] = out


_final_call = pl.pallas_call(
    _final_body,
    out_shape=jax.ShapeDtypeStruct((1, G), jnp.float32),
)


def kernel(x, edge_index, batch_index, W1, b1, Wout, bout):
  x = x.astype(jnp.float32)
  src = edge_index[0].astype(jnp.int32)
  dst = edge_index[1].astype(jnp.int32)
  npad = E_PAD - E
  pad_ids = jnp.arange(npad, dtype=jnp.int32)
  pad_src = pad_ids % N
  pad_dst = N + pad_ids % (NP - N)
  srcp = jnp.concatenate([src, pad_src])
  dstp = jnp.concatenate([dst, pad_dst])
  pk = dstp * 16384 + srcp

  degp = _get_deg_kernel()(pk).reshape(NW, NP)
  yt, dinv = _scale_call(x, W1.astype(jnp.float32), degp)
  p = _get_msg_kernel()(pk, yt)
  p2 = p.reshape(NQ * H, NP)

  bi = batch_index.astype(jnp.int32).reshape(1, N)
  out = _final_call(p2, yt, dinv, bi,
                    b1.astype(jnp.float32).reshape(H, 1),
                    Wout.astype(jnp.float32).reshape(1, 2 * H),
                    bout.astype(jnp.float32).reshape(1, 1))
  return out.T
